# baseline (speedup 1.0000x reference)
import jax
import jax.numpy as jnp
from jax import lax
from jax.experimental import pallas as pl
from jax.experimental.pallas import tpu as pltpu
from jax.experimental.pallas import tpu_sc as plsc

_NC = 2
_NS = 16
_NW = _NC * _NS


def _make_gather(num_emb, feat, b_total):
    assert b_total % (8 * _NW) == 0
    b_per_w = b_total // _NW
    chunk = 1664
    assert b_per_w % chunk == 0
    nchunk = b_per_w // chunk

    mesh = plsc.VectorSubcoreMesh(
        core_axis_name="c", subcore_axis_name="s",
        num_cores=_NC, num_subcores=_NS)

    @jax.jit
    def call(table, idx):
        @pl.kernel(
            out_type=jax.ShapeDtypeStruct((b_total, feat), jnp.float32),
            mesh=mesh,
            compiler_params=pltpu.CompilerParams(use_tc_tiling_on_sc=False),
            scratch_types=[
                pltpu.VMEM((b_per_w,), jnp.int32),
                pltpu.VMEM((chunk, feat), jnp.float32),
                pltpu.VMEM((chunk, feat), jnp.float32),
                pltpu.SemaphoreType.DMA,
                pltpu.SemaphoreType.DMA,
                pltpu.SemaphoreType.DMA,
                pltpu.SemaphoreType.DMA,
            ],
        )
        def k(table_hbm, idx_hbm, out_hbm, idx_v, buf0, buf1,
              gsem0, gsem1, osem0, osem1):
            wid = lax.axis_index("s") * _NC + lax.axis_index("c")
            base = wid * b_per_w
            pltpu.sync_copy(idx_hbm.at[pl.ds(base, b_per_w)], idx_v)

            bufs = (buf0, buf1)
            gsems = (gsem0, gsem1)
            osems = (osem0, osem1)
            gcp = [None, None]
            ocp = [None, None]

            gcp[0] = pltpu.async_copy(
                table_hbm.at[idx_v.at[pl.ds(0, chunk)]], bufs[0], gsems[0])
            for c in range(nchunk):
                b = c % 2
                nb = (c + 1) % 2
                if c + 1 < nchunk:
                    if c + 1 >= 2:
                        ocp[nb].wait()
                    gcp[nb] = pltpu.async_copy(
                        table_hbm.at[idx_v.at[pl.ds((c + 1) * chunk, chunk)]],
                        bufs[nb], gsems[nb])
                gcp[b].wait()
                ocp[b] = pltpu.async_copy(
                    bufs[b], out_hbm.at[pl.ds(base + c * chunk, chunk)],
                    osems[b])
            ocp[(nchunk - 2) % 2].wait()
            ocp[(nchunk - 1) % 2].wait()

        return k(table, idx)

    return call


def kernel(inputs, embedding):
    batch, fields = inputs.shape
    num_emb, feat = embedding.shape
    b_total = batch * fields
    idx = inputs.reshape(b_total).astype(jnp.int32)
    call = _make_gather(num_emb, feat, b_total)
    out = call(embedding, idx)
    return out.reshape(batch, fields, feat)


# speedup vs baseline: 1.4771x; 1.4771x over previous
import functools

import jax
import jax.numpy as jnp
from jax import lax
from jax.experimental import pallas as pl
from jax.experimental.pallas import tpu as pltpu
from jax.experimental.pallas import tpu_sc as plsc

_NC = 2
_NS = 16
_NW = _NC * _NS
_L = 16

_CHUNK = 128
_ROWS_PER_PACK = 8


def _make_lookup(batch, fields, feat, n_packed):
    assert batch % _NW == 0
    b_per_w = batch // _NW
    nq = b_per_w // _CHUNK
    assert b_per_w % _CHUNK == 0
    nu = fields * nq

    mesh = plsc.VectorSubcoreMesh(
        core_axis_name="c", subcore_axis_name="s",
        num_cores=_NC, num_subcores=_NS)

    @pl.kernel(
        out_type=jax.ShapeDtypeStruct((fields, feat, batch), jnp.float32),
        mesh=mesh,
        compiler_params=pltpu.CompilerParams(needs_layout_passes=False),
        scratch_types=[
            pltpu.VMEM((_CHUNK,), jnp.int32),
            pltpu.VMEM((_CHUNK,), jnp.int32),
            pltpu.VMEM((_CHUNK,), jnp.int32),
            pltpu.VMEM((_CHUNK, 128), jnp.float32),
            pltpu.VMEM((_CHUNK, 128), jnp.float32),
            pltpu.VMEM((_CHUNK,), jnp.int32),
            pltpu.VMEM((_CHUNK,), jnp.int32),
            pltpu.VMEM((feat, b_per_w), jnp.float32),
            pltpu.SemaphoreType.DMA,
            pltpu.SemaphoreType.DMA,
        ],
    )
    def k(packed_hbm, idx_hbm, out_hbm, idx_v, pidx0, pidx1, g0, g1,
          off0, off1, o_tile, sem0, sem1):
        wid = lax.axis_index("s") * _NC + lax.axis_index("c")
        b0 = wid * b_per_w
        iota = lax.iota(jnp.int32, _L)

        pidx = (pidx0, pidx1)
        gbuf = (g0, g1)
        offb = (off0, off1)
        sems = (sem0, sem1)

        def stage(u, par):
            f = u // nq
            q = u % nq
            src = f * batch + b0 + q * _CHUNK
            pltpu.sync_copy(idx_hbm.at[pl.ds(src, _CHUNK)], idx_v)
            for t in range(_CHUNK // _L):
                v = idx_v[pl.ds(t * _L, _L)]
                pidx[par][pl.ds(t * _L, _L)] = lax.shift_right_logical(v, 3)
                offb[par][pl.ds(t * _L, _L)] = lax.shift_left(
                    jnp.bitwise_and(v, 7), 4)
            return pltpu.async_copy(
                packed_hbm.at[pidx[par]], gbuf[par], sems[par])

        def gwait(par):
            pltpu.make_async_copy(
                packed_hbm.at[pidx[par]], gbuf[par], sems[par]).wait()

        def extract(u, par):
            q = u % nq
            col0 = q * _CHUNK
            g = gbuf[par]
            off_ref = offb[par]
            for t in range(_CHUNK // _L):
                rows = jnp.full((_L,), t * _L, jnp.int32) + iota
                off = off_ref[pl.ds(t * _L, _L)]
                for j in range(feat):
                    vals = plsc.load_gather(g, [rows, off + j])
                    o_tile[j, pl.ds(col0 + t * _L, _L)] = vals

        def flush(u):
            f = u // nq
            pltpu.sync_copy(
                o_tile, out_hbm.at[f, :, pl.ds(b0, b_per_w)])

        cp0 = stage(0, 0)

        def body(s, carry):
            u = s * 2
            cpn = stage(u + 1, 1)
            gwait(0)
            extract(u, 0)

            @pl.when((u % nq) == (nq - 1))
            def _():
                flush(u)

            @pl.when(s < (nu // 2 - 1))
            def _():
                stage(u + 2, 0)
            gwait(1)
            extract(u + 1, 1)

            @pl.when(((u + 1) % nq) == (nq - 1))
            def _():
                flush(u + 1)

            return carry

        del cp0
        lax.fori_loop(0, nu // 2, body, 0)

    return k


def kernel(inputs, embedding):
    batch, fields = inputs.shape
    num_emb, feat = embedding.shape
    assert num_emb % _ROWS_PER_PACK == 0
    packed = jnp.reshape(embedding, (num_emb // _ROWS_PER_PACK, 128))
    idx_fm = jnp.transpose(inputs).reshape(batch * fields).astype(jnp.int32)
    call = _make_lookup(batch, fields, feat, num_emb // _ROWS_PER_PACK)
    out_t = call(packed, idx_fm)
    return jnp.transpose(out_t, (2, 0, 1))
